# baseline (speedup 1.0000x reference)
import functools
import math

import jax
import jax.numpy as jnp
from jax import lax
from jax.experimental import pallas as pl
from jax.experimental.pallas import tpu as pltpu

_SLOPE = 0.2
_ISQ2 = 1.0 / math.sqrt(2.0)
_VMEM = 64 * 1024 * 1024


def _lrelu(v):
    return jnp.where(v >= 0, v, _SLOPE * v)


def _conv3x3(x, w, b, mask, wp):
    cin, L = x.shape
    z = jnp.zeros((cin, wp + 1), jnp.float32)
    ext = jnp.concatenate([z, x, z], axis=1)
    col = jnp.concatenate(
        [ext[:, a * wp + c: a * wp + c + L] for a in range(3) for c in range(3)],
        axis=0)
    acc = lax.dot_general(w, col, (((1,), (0,)), ((), ())),
                          preferred_element_type=jnp.float32)
    return (acc + b) * mask


_TAP = ((1, -1), (0, 0), (1, 0))


def _dw3x3s2(ph, wdw, bdw, mask, wp2):
    c, L2 = ph.shape[1:]
    z = jnp.zeros((c, wp2 + 1), jnp.float32)
    ext = [jnp.concatenate([z, ph[q], z], axis=1) for q in range(4)]
    acc = jnp.zeros((c, L2), jnp.float32)
    for a in range(3):
        pa, da = _TAP[a]
        for b in range(3):
            pb, db = _TAP[b]
            off = (da + 1) * wp2 + (db + 1)
            acc = acc + wdw[a * 3 + b] * ext[2 * pa + pb][:, off: off + L2]
    return (acc + bdw) * mask


def _stem_kernel(x_ref, w0_ref, b0_ref, w1_ref, b1_ref, mask_ref,
                 h0_ref, c1_ref, *, wp):
    mask = mask_ref[...]
    xe = jnp.concatenate([jnp.zeros((1, wp + 1), jnp.float32), x_ref[0],
                          jnp.zeros((1, wp + 1), jnp.float32)], axis=1)
    L = mask.shape[1]
    w0 = w0_ref[...]
    acc = jnp.zeros((w0.shape[0], L), jnp.float32)
    for t in range(9):
        off = (t // 3) * wp + t % 3
        acc = acc + w0[:, t:t + 1] * xe[:, off: off + L]
    h0 = (acc + b0_ref[...]) * mask
    h0_ref[0] = h0
    c1_ref[0] = _conv3x3(_lrelu(h0), w1_ref[...], b1_ref[...], mask, wp)


def _blk_kernel(ph_ref, pool_ref, wdw_ref, bdw_ref, w2_ref, b2_ref,
                wsc_ref, w1n_ref, b1n_ref, mask_ref, m_ref, c1n_ref,
                *, wp2, learned_sc):
    mask = mask_ref[...]
    d = _dw3x3s2(ph_ref[0], wdw_ref[...], bdw_ref[...], mask, wp2)
    r = _conv3x3(_lrelu(d), w2_ref[...], b2_ref[...], mask, wp2)
    pool = pool_ref[0]
    if learned_sc:
        sc = lax.dot_general(wsc_ref[...], pool, (((1,), (0,)), ((), ())),
                             preferred_element_type=jnp.float32)
    else:
        sc = pool
    m = (sc + r) * _ISQ2
    m_ref[0] = m
    c1n_ref[0] = _conv3x3(_lrelu(m), w1n_ref[...], b1n_ref[...], mask, wp2)


def _head_kernel(ph_ref, pool_ref, wdw_ref, bdw_ref, w2_ref, b2_ref,
                 w5_ref, b5_ref, wl_ref, bl_ref, mask_ref, out_ref, *, wp2):
    mask = mask_ref[...]
    d = _dw3x3s2(ph_ref[0], wdw_ref[...], bdw_ref[...], mask, wp2)
    r = _conv3x3(_lrelu(d), w2_ref[...], b2_ref[...], mask, wp2)
    m = (pool_ref[0] + r) * _ISQ2
    a = _lrelu(m)
    cdim, L = a.shape
    ext = jnp.concatenate([a, jnp.zeros((cdim, 4 * wp2 + 5), jnp.float32)],
                          axis=1)
    col = jnp.concatenate(
        [ext[:, rr * wp2 + ss: rr * wp2 + ss + L]
         for rr in range(5) for ss in range(5)], axis=0)
    c5 = lax.dot_general(w5_ref[...], col, (((1,), (0,)), ((), ())),
                         preferred_element_type=jnp.float32) + b5_ref[...]
    out_ref[0] = lax.dot_general(wl_ref[...], _lrelu(c5),
                                 (((1,), (0,)), ((), ())),
                                 preferred_element_type=jnp.float32) + bl_ref[...]


def _full(shape):
    shape = tuple(shape)
    return pl.BlockSpec(shape, lambda i: (0,) * len(shape))


def _batched(shape):
    shape = tuple(shape)
    return pl.BlockSpec((1,) + shape, lambda i: (i,) + (0,) * len(shape))


_CP = pltpu.CompilerParams(dimension_semantics=("parallel",),
                           vmem_limit_bytes=_VMEM)


def _mask_flat(H, W, nb):
    m = jnp.pad(jnp.ones((H, W), jnp.float32), ((1, 1), (1, 1)))
    return jnp.tile(m.reshape(1, (H + 2) * (W + 2)), (1, nb))


def _prep_w(w, b):
    cin, cout = w.shape[2], w.shape[3]
    return w.reshape(9 * cin, cout).T, b.reshape(cout, 1)


def _to_lane_batched(x4, gb, nb):
    B, C, Hp, Wp = x4.shape
    v = x4.reshape(gb, nb, C, Hp * Wp).transpose(0, 2, 1, 3)
    return v.reshape(gb, C, nb * Hp * Wp)


def _from_lane_batched(flat, nb, Hp, Wp):
    gb, C = flat.shape[:2]
    v = flat.reshape(gb, C, nb, Hp, Wp).transpose(0, 2, 1, 3, 4)
    return v.reshape(gb * nb, C, Hp, Wp)


def _phases(y4):
    B, C, Hp, Wp = y4.shape
    H, W = Hp - 2, Wp - 2
    inner = y4[:, :, 1:H + 1, 1:W + 1]
    phs = [jnp.pad(inner[:, :, pa::2, pb::2], ((0, 0), (0, 0), (1, 1), (1, 1)))
           for pa in (0, 1) for pb in (0, 1)]
    return jnp.stack(phs, axis=1)


def _pool2(y4):
    B, C, Hp, Wp = y4.shape
    H, W = Hp - 2, Wp - 2
    inner = y4[:, :, 1:H + 1, 1:W + 1]
    p = inner.reshape(B, C, H // 2, 2, W // 2, 2).mean(axis=(3, 5))
    return jnp.pad(p, ((0, 0), (0, 0), (1, 1), (1, 1)))


def kernel(x, conv0_w, conv0_b, conv5_w, conv5_b, lin_w, lin_b,
           b0_conv1_w, b0_conv1_b, b0_dw_w, b0_dw_b, b0_conv2_w, b0_conv2_b, b0_conv1x1_w,
           b1_conv1_w, b1_conv1_b, b1_dw_w, b1_dw_b, b1_conv2_w, b1_conv2_b, b1_conv1x1_w,
           b2_conv1_w, b2_conv1_b, b2_dw_w, b2_dw_b, b2_conv2_w, b2_conv2_b,
           b3_conv1_w, b3_conv1_b, b3_dw_w, b3_dw_b, b3_conv2_w, b3_conv2_b):
    B = x.shape[0]
    x = x.astype(jnp.float32)
    blocks = [
        dict(w1=b0_conv1_w, b1=b0_conv1_b, wd=b0_dw_w, bd=b0_dw_b,
             w2=b0_conv2_w, b2=b0_conv2_b, sc=b0_conv1x1_w),
        dict(w1=b1_conv1_w, b1=b1_conv1_b, wd=b1_dw_w, bd=b1_dw_b,
             w2=b1_conv2_w, b2=b1_conv2_b, sc=b1_conv1x1_w),
        dict(w1=b2_conv1_w, b1=b2_conv1_b, wd=b2_dw_w, bd=b2_dw_b,
             w2=b2_conv2_w, b2=b2_conv2_b, sc=None),
        dict(w1=b3_conv1_w, b1=b3_conv1_b, wd=b3_dw_w, bd=b3_dw_b,
             w2=b3_conv2_w, b2=b3_conv2_b, sc=None),
    ]

    H, W = 80, 80
    NBs = 4
    gb = B // NBs
    Lp, wp = (H + 2) * (W + 2), W + 2
    xc = jnp.pad(x, ((0, 0), (0, 0), (1, 1), (1, 1))).reshape(B, 1, Lp)
    xc = _to_lane_batched(xc.reshape(B, 1, H + 2, W + 2), gb, NBs)
    w0 = conv0_w.reshape(9, conv0_w.shape[3]).T
    b0 = conv0_b.reshape(-1, 1)
    w1, b1 = _prep_w(blocks[0]['w1'], blocks[0]['b1'])
    d0 = w0.shape[0]
    mask = _mask_flat(H, W, NBs)
    h0f, c1f = pl.pallas_call(
        functools.partial(_stem_kernel, wp=wp),
        out_shape=(jax.ShapeDtypeStruct((gb, d0, NBs * Lp), jnp.float32),
                   jax.ShapeDtypeStruct((gb, d0, NBs * Lp), jnp.float32)),
        grid=(gb,),
        in_specs=[_batched((1, NBs * Lp)), _full(w0.shape), _full(b0.shape),
                  _full(w1.shape), _full(b1.shape), _full(mask.shape)],
        out_specs=(_batched((d0, NBs * Lp)), _batched((d0, NBs * Lp))),
        compiler_params=_CP,
    )(xc, w0, b0, w1, b1, mask)

    m4 = _from_lane_batched(h0f, NBs, H + 2, W + 2)
    c14 = _from_lane_batched(c1f, NBs, H + 2, W + 2)

    for k, blk in enumerate(blocks):
        H2, W2 = H // 2, W // 2
        Lp2, wp2 = (H2 + 2) * (W2 + 2), W2 + 2
        NB = 16 if k < 2 else 32
        gb = B // NB
        ph = _phases(c14)
        cin = ph.shape[2]
        ph = ph.reshape(gb, NB, 4, cin, Lp2).transpose(0, 2, 3, 1, 4)
        ph = ph.reshape(gb, 4, cin, NB * Lp2)
        pool = _to_lane_batched(_pool2(m4), gb, NB)
        wdw = blk['wd'].reshape(9, cin, 1)
        bdw = blk['bd'].reshape(cin, 1)
        w2, b2 = _prep_w(blk['w2'], blk['b2'])
        cout = w2.shape[0]
        learned = blk['sc'] is not None
        wsc = blk['sc'][0, 0].T if learned else jnp.zeros((cout, cin), jnp.float32)
        mask2 = _mask_flat(H2, W2, NB)

        if k + 1 < len(blocks):
            nxt = blocks[k + 1]
            w1n, b1n = _prep_w(nxt['w1'], nxt['b1'])
            mf, c1nf = pl.pallas_call(
                functools.partial(_blk_kernel, wp2=wp2, learned_sc=learned),
                out_shape=(jax.ShapeDtypeStruct((gb, cout, NB * Lp2), jnp.float32),
                           jax.ShapeDtypeStruct((gb, cout, NB * Lp2), jnp.float32)),
                grid=(gb,),
                in_specs=[_batched((4, cin, NB * Lp2)), _batched((cin, NB * Lp2)),
                          _full(wdw.shape), _full(bdw.shape),
                          _full(w2.shape), _full(b2.shape), _full(wsc.shape),
                          _full(w1n.shape), _full(b1n.shape), _full(mask2.shape)],
                out_specs=(_batched((cout, NB * Lp2)), _batched((cout, NB * Lp2))),
                compiler_params=_CP,
            )(ph, pool, wdw, bdw, w2, b2, wsc, w1n, b1n, mask2)
            m4 = _from_lane_batched(mf, NB, H2 + 2, W2 + 2)
            c14 = _from_lane_batched(c1nf, NB, H2 + 2, W2 + 2)
        else:
            w5 = conv5_w.reshape(25 * cout, cout).T
            b5 = conv5_b.reshape(cout, 1)
            wl = lin_w
            bl = lin_b.reshape(-1, 1)
            S = wl.shape[0]
            outf = pl.pallas_call(
                functools.partial(_head_kernel, wp2=wp2),
                out_shape=jax.ShapeDtypeStruct((gb, S, NB * Lp2), jnp.float32),
                grid=(gb,),
                in_specs=[_batched((4, cin, NB * Lp2)), _batched((cin, NB * Lp2)),
                          _full(wdw.shape), _full(bdw.shape),
                          _full(w2.shape), _full(b2.shape),
                          _full(w5.shape), _full(b5.shape),
                          _full(wl.shape), _full(bl.shape), _full(mask2.shape)],
                out_specs=_batched((S, NB * Lp2)),
                compiler_params=_CP,
            )(ph, pool, wdw, bdw, w2, b2, w5, b5, wl, bl, mask2)
            out = outf.reshape(gb, S, NB, Lp2)[:, :, :, wp2 + 1]
            return out.transpose(0, 2, 1).reshape(B, S)
        H, W = H2, W2
    raise AssertionError("unreachable")


# speedup vs baseline: 22.7211x; 22.7211x over previous
import functools
import math

import jax
import jax.numpy as jnp
from jax import lax
from jax.experimental import pallas as pl
from jax.experimental.pallas import tpu as pltpu

_SLOPE = 0.2
_ISQ2 = 1.0 / math.sqrt(2.0)
_VMEM = 64 * 1024 * 1024


def _lrelu(v):
    return jnp.where(v >= 0, v, _SLOPE * v)


def _conv3x3(x, w, b, mask, wp):
    cin, L = x.shape
    z = jnp.zeros((cin, wp + 1), jnp.float32)
    ext = jnp.concatenate([z, x, z], axis=1)
    col = jnp.concatenate(
        [ext[:, a * wp + c: a * wp + c + L] for a in range(3) for c in range(3)],
        axis=0)
    acc = lax.dot_general(w, col, (((1,), (0,)), ((), ())),
                          preferred_element_type=jnp.float32)
    return (acc + b) * mask


_TAP = ((1, -1), (0, 0), (1, 0))


def _dw3x3s2(ph, wdw, bdw, mask, wp2):
    c, L2 = ph.shape[1:]
    z = jnp.zeros((c, wp2 + 1), jnp.float32)
    ext = [jnp.concatenate([z, ph[q], z], axis=1) for q in range(4)]
    acc = jnp.zeros((c, L2), jnp.float32)
    for a in range(3):
        pa, da = _TAP[a]
        for b in range(3):
            pb, db = _TAP[b]
            off = (da + 1) * wp2 + (db + 1)
            acc = acc + wdw[a * 3 + b] * ext[2 * pa + pb][:, off: off + L2]
    return (acc + bdw) * mask


def _stem_kernel(x_ref, w0_ref, b0_ref, w1_ref, b1_ref, mask_ref,
                 h0_ref, c1_ref, *, wp):
    mask = mask_ref[...]
    xe = jnp.concatenate([jnp.zeros((1, wp + 1), jnp.float32), x_ref[0],
                          jnp.zeros((1, wp + 1), jnp.float32)], axis=1)
    L = mask.shape[1]
    w0 = w0_ref[...]
    acc = jnp.zeros((w0.shape[0], L), jnp.float32)
    for t in range(9):
        off = (t // 3) * wp + t % 3
        acc = acc + w0[:, t:t + 1] * xe[:, off: off + L]
    h0 = (acc + b0_ref[...]) * mask
    h0_ref[0] = h0
    c1_ref[0] = _conv3x3(_lrelu(h0), w1_ref[...], b1_ref[...], mask, wp)


def _blk_kernel(ph_ref, pool_ref, wdw_ref, bdw_ref, w2_ref, b2_ref,
                wsc_ref, w1n_ref, b1n_ref, mask_ref, m_ref, c1n_ref,
                *, wp2, learned_sc):
    mask = mask_ref[...]
    d = _dw3x3s2(ph_ref[0], wdw_ref[...], bdw_ref[...], mask, wp2)
    r = _conv3x3(_lrelu(d), w2_ref[...], b2_ref[...], mask, wp2)
    pool = pool_ref[0]
    if learned_sc:
        sc = lax.dot_general(wsc_ref[...], pool, (((1,), (0,)), ((), ())),
                             preferred_element_type=jnp.float32)
    else:
        sc = pool
    m = (sc + r) * _ISQ2
    m_ref[0] = m
    c1n_ref[0] = _conv3x3(_lrelu(m), w1n_ref[...], b1n_ref[...], mask, wp2)


def _head_kernel(ph_ref, pool_ref, wdw_ref, bdw_ref, w2_ref, b2_ref,
                 w5_ref, b5_ref, wl_ref, bl_ref, mask_ref, out_ref, *, wp2):
    mask = mask_ref[...]
    d = _dw3x3s2(ph_ref[0], wdw_ref[...], bdw_ref[...], mask, wp2)
    r = _conv3x3(_lrelu(d), w2_ref[...], b2_ref[...], mask, wp2)
    m = (pool_ref[0] + r) * _ISQ2
    a = _lrelu(m)
    cdim, L = a.shape
    ext = jnp.concatenate([a, jnp.zeros((cdim, 4 * wp2 + 5), jnp.float32)],
                          axis=1)
    col = jnp.concatenate(
        [ext[:, rr * wp2 + ss: rr * wp2 + ss + L]
         for rr in range(5) for ss in range(5)], axis=0)
    c5 = lax.dot_general(w5_ref[...], col, (((1,), (0,)), ((), ())),
                         preferred_element_type=jnp.float32) + b5_ref[...]
    out_ref[0] = lax.dot_general(wl_ref[...], _lrelu(c5),
                                 (((1,), (0,)), ((), ())),
                                 preferred_element_type=jnp.float32) + bl_ref[...]


def _full(shape):
    shape = tuple(shape)
    return pl.BlockSpec(shape, lambda i: (0,) * len(shape))


def _batched(shape):
    shape = tuple(shape)
    return pl.BlockSpec((1,) + shape, lambda i: (i,) + (0,) * len(shape))


_CP = pltpu.CompilerParams(dimension_semantics=("parallel",),
                           vmem_limit_bytes=_VMEM)


def _mask_flat(H, W, nb):
    m = jnp.pad(jnp.ones((H, W), jnp.float32), ((1, 1), (1, 1)))
    return jnp.tile(m.reshape(1, (H + 2) * (W + 2)), (1, nb))


def _prep_w(w, b):
    cin, cout = w.shape[2], w.shape[3]
    return w.reshape(9 * cin, cout).T, b.reshape(cout, 1)


def _phases5(flat, nb, Hp, Wp):
    gb, C = flat.shape[:2]
    H, W = Hp - 2, Wp - 2
    y = flat.reshape(gb, C, nb, Hp, Wp)
    inner = y[:, :, :, 1:H + 1, 1:W + 1]
    phs = []
    for pa in (0, 1):
        for pb in (0, 1):
            p = jnp.pad(inner[:, :, :, pa::2, pb::2],
                        ((0, 0), (0, 0), (0, 0), (1, 1), (1, 1)))
            phs.append(p.reshape(gb, C, nb * (H // 2 + 2) * (W // 2 + 2)))
    return jnp.stack(phs, axis=1)


def _pool5(flat, nb, Hp, Wp):
    gb, C = flat.shape[:2]
    H, W = Hp - 2, Wp - 2
    y = flat.reshape(gb, C, nb, Hp, Wp)
    inner = y[:, :, :, 1:H + 1, 1:W + 1]
    p = inner.reshape(gb, C, nb, H // 2, 2, W // 2, 2).mean(axis=(4, 6))
    p = jnp.pad(p, ((0, 0), (0, 0), (0, 0), (1, 1), (1, 1)))
    return p.reshape(gb, C, nb * (H // 2 + 2) * (W // 2 + 2))


def kernel(x, conv0_w, conv0_b, conv5_w, conv5_b, lin_w, lin_b,
           b0_conv1_w, b0_conv1_b, b0_dw_w, b0_dw_b, b0_conv2_w, b0_conv2_b, b0_conv1x1_w,
           b1_conv1_w, b1_conv1_b, b1_dw_w, b1_dw_b, b1_conv2_w, b1_conv2_b, b1_conv1x1_w,
           b2_conv1_w, b2_conv1_b, b2_dw_w, b2_dw_b, b2_conv2_w, b2_conv2_b,
           b3_conv1_w, b3_conv1_b, b3_dw_w, b3_dw_b, b3_conv2_w, b3_conv2_b):
    B = x.shape[0]
    x = x.astype(jnp.float32)
    blocks = [
        dict(w1=b0_conv1_w, b1=b0_conv1_b, wd=b0_dw_w, bd=b0_dw_b,
             w2=b0_conv2_w, b2=b0_conv2_b, sc=b0_conv1x1_w),
        dict(w1=b1_conv1_w, b1=b1_conv1_b, wd=b1_dw_w, bd=b1_dw_b,
             w2=b1_conv2_w, b2=b1_conv2_b, sc=b1_conv1x1_w),
        dict(w1=b2_conv1_w, b1=b2_conv1_b, wd=b2_dw_w, bd=b2_dw_b,
             w2=b2_conv2_w, b2=b2_conv2_b, sc=None),
        dict(w1=b3_conv1_w, b1=b3_conv1_b, wd=b3_dw_w, bd=b3_dw_b,
             w2=b3_conv2_w, b2=b3_conv2_b, sc=None),
    ]

    H, W = 80, 80
    NB = 8
    gb = B // NB
    Lp, wp = (H + 2) * (W + 2), W + 2
    xc = jnp.pad(x, ((0, 0), (0, 0), (1, 1), (1, 1)))
    xc = xc.reshape(gb, NB, 1, Lp).transpose(0, 2, 1, 3).reshape(gb, 1, NB * Lp)
    w0 = conv0_w.reshape(9, conv0_w.shape[3]).T
    b0 = conv0_b.reshape(-1, 1)
    w1, b1 = _prep_w(blocks[0]['w1'], blocks[0]['b1'])
    d0 = w0.shape[0]
    mask = _mask_flat(H, W, NB)
    h0f, c1f = pl.pallas_call(
        functools.partial(_stem_kernel, wp=wp),
        out_shape=(jax.ShapeDtypeStruct((gb, d0, NB * Lp), jnp.float32),
                   jax.ShapeDtypeStruct((gb, d0, NB * Lp), jnp.float32)),
        grid=(gb,),
        in_specs=[_batched((1, NB * Lp)), _full(w0.shape), _full(b0.shape),
                  _full(w1.shape), _full(b1.shape), _full(mask.shape)],
        out_specs=(_batched((d0, NB * Lp)), _batched((d0, NB * Lp))),
        compiler_params=_CP,
    )(xc, w0, b0, w1, b1, mask)

    mflat, c1flat = h0f, c1f

    for k, blk in enumerate(blocks):
        H2, W2 = H // 2, W // 2
        Lp2, wp2 = (H2 + 2) * (W2 + 2), W2 + 2
        ph = _phases5(c1flat, NB, H + 2, W + 2)
        cin = ph.shape[2]
        pool = _pool5(mflat, NB, H + 2, W + 2)
        wdw = blk['wd'].reshape(9, cin, 1)
        bdw = blk['bd'].reshape(cin, 1)
        w2, b2 = _prep_w(blk['w2'], blk['b2'])
        cout = w2.shape[0]
        learned = blk['sc'] is not None
        wsc = blk['sc'][0, 0].T if learned else jnp.zeros((cout, cin), jnp.float32)
        mask2 = _mask_flat(H2, W2, NB)

        if k + 1 < len(blocks):
            nxt = blocks[k + 1]
            w1n, b1n = _prep_w(nxt['w1'], nxt['b1'])
            mf, c1nf = pl.pallas_call(
                functools.partial(_blk_kernel, wp2=wp2, learned_sc=learned),
                out_shape=(jax.ShapeDtypeStruct((gb, cout, NB * Lp2), jnp.float32),
                           jax.ShapeDtypeStruct((gb, cout, NB * Lp2), jnp.float32)),
                grid=(gb,),
                in_specs=[_batched((4, cin, NB * Lp2)), _batched((cin, NB * Lp2)),
                          _full(wdw.shape), _full(bdw.shape),
                          _full(w2.shape), _full(b2.shape), _full(wsc.shape),
                          _full(w1n.shape), _full(b1n.shape), _full(mask2.shape)],
                out_specs=(_batched((cout, NB * Lp2)), _batched((cout, NB * Lp2))),
                compiler_params=_CP,
            )(ph, pool, wdw, bdw, w2, b2, wsc, w1n, b1n, mask2)
            mflat, c1flat = mf, c1nf
        else:
            w5 = conv5_w.reshape(25 * cout, cout).T
            b5 = conv5_b.reshape(cout, 1)
            wl = lin_w
            bl = lin_b.reshape(-1, 1)
            S = wl.shape[0]
            outf = pl.pallas_call(
                functools.partial(_head_kernel, wp2=wp2),
                out_shape=jax.ShapeDtypeStruct((gb, S, NB * Lp2), jnp.float32),
                grid=(gb,),
                in_specs=[_batched((4, cin, NB * Lp2)), _batched((cin, NB * Lp2)),
                          _full(wdw.shape), _full(bdw.shape),
                          _full(w2.shape), _full(b2.shape),
                          _full(w5.shape), _full(b5.shape),
                          _full(wl.shape), _full(bl.shape), _full(mask2.shape)],
                out_specs=_batched((S, NB * Lp2)),
                compiler_params=_CP,
            )(ph, pool, wdw, bdw, w2, b2, w5, b5, wl, bl, mask2)
            out = outf.reshape(gb, S, NB, Lp2)[:, :, :, wp2 + 1]
            return out.transpose(0, 2, 1).reshape(B, S)
        H, W = H2, W2
    raise AssertionError("unreachable")
